# baseline (speedup 1.0000x reference)
import functools

import jax
import jax.numpy as jnp
from jax import lax
from jax.experimental import pallas as pl
from jax.experimental.pallas import tpu as pltpu
from jax.experimental.pallas import tpu_sc as plsc

_N = 10000
_E = 320000
_D = 128
_C = 16

_NC = 2
_NS = 16
_NW = _NC * _NS
_K = 128
_CPT = 80
_EPAD = _NW * _CPT * _K
_NP = 10240
_TRASH = 10200
_RPT = _NP // _NS
_NBUF = 2
_J = 8
_NJ = _CPT // _J

_mesh = plsc.VectorSubcoreMesh(
    core_axis_name="c", subcore_axis_name="s", num_cores=_NC, num_subcores=_NS
)


def _load_block(src_hbm, dst_hbm, si2, di2, bsems, wid, j, sl):
    j0 = pl.multiple_of(j * _J, 8)
    pltpu.async_copy(src_hbm.at[wid].at[pl.ds(j0, _J)], si2.at[sl], bsems[sl])
    pltpu.async_copy(dst_hbm.at[wid].at[pl.ds(j0, _J)], di2.at[sl], bsems[sl])


def _wait_block(src_hbm, dst_hbm, si2, di2, bsems, wid, j, sl):
    j0 = pl.multiple_of(j * _J, 8)
    pltpu.make_async_copy(
        src_hbm.at[wid].at[pl.ds(j0, _J)], si2.at[sl], bsems[sl]).wait()
    pltpu.make_async_copy(
        dst_hbm.at[wid].at[pl.ds(j0, _J)], di2.at[sl], bsems[sl]).wait()


@functools.partial(
    pl.kernel,
    out_type=jax.ShapeDtypeStruct((_NC, _NP, _D), jnp.float32),
    mesh=_mesh,
    scratch_types=[
        pltpu.VMEM((2, _J, _K), jnp.int32),
        pltpu.VMEM((2, _J, _K), jnp.int32),
        pltpu.VMEM((_K, _D), jnp.float32),
        pltpu.VMEM((_K, _D), jnp.float32),
        pltpu.VMEM_SHARED((_NP, _D), jnp.float32),
        [pltpu.SemaphoreType.DMA] * 2,
        pltpu.SemaphoreType.DMA,
    ],
)
def _degree_kernel(src_hbm, dst_hbm, zeros_hbm, ones_s_hbm, ones_d_hbm,
                   out_hbm, si2, di2, ones_s, ones_d, acc, bsems, ssem):
    c = lax.axis_index("c")
    s = lax.axis_index("s")
    wid = c * _NS + s
    r0 = s * _RPT
    pltpu.sync_copy(ones_s_hbm, ones_s)
    pltpu.sync_copy(ones_d_hbm, ones_d)
    pltpu.sync_copy(zeros_hbm.at[pl.ds(r0, _RPT)], acc.at[pl.ds(r0, _RPT)])
    plsc.subcore_barrier()

    def emit_block(j, p, wait_next, load_next):
        for q in range(_J):
            pltpu.async_copy(ones_s, acc.at[si2.at[p].at[q]], ssem, add=True)
            pltpu.async_copy(ones_d, acc.at[di2.at[p].at[q]], ssem, add=True)
        for q in range(2 * _J):
            pltpu.make_async_copy(ones_s, acc.at[si2.at[p].at[0]],
                                  ssem).wait()
        if wait_next:
            _wait_block(src_hbm, dst_hbm, si2, di2, bsems, wid, j + 1, 1 - p)
        if load_next:
            _load_block(src_hbm, dst_hbm, si2, di2, bsems, wid, j + 2, p)

    _load_block(src_hbm, dst_hbm, si2, di2, bsems, wid, 0, 0)
    _load_block(src_hbm, dst_hbm, si2, di2, bsems, wid, 1, 1)
    _wait_block(src_hbm, dst_hbm, si2, di2, bsems, wid, 0, 0)

    def body(j0, carry):
        emit_block(j0 * 2, 0, True, True)
        emit_block(j0 * 2 + 1, 1, True, True)
        return carry

    lax.fori_loop(0, (_NJ - 2) // 2, body, 0)
    emit_block(_NJ - 2, 0, True, False)
    emit_block(_NJ - 1, 1, False, False)

    plsc.subcore_barrier()
    pltpu.sync_copy(acc.at[pl.ds(r0, _RPT)],
                    out_hbm.at[c].at[pl.ds(r0, _RPT)])


@functools.partial(
    pl.kernel,
    out_type=jax.ShapeDtypeStruct((_NC, _NP, _D), jnp.float32),
    mesh=_mesh,
    scratch_types=[
        pltpu.VMEM((_K,), jnp.int32),
        pltpu.VMEM((_K,), jnp.int32),
        pltpu.VMEM((1, _K), jnp.int32),
        pltpu.VMEM((1, _K), jnp.int32),
        pltpu.VMEM((_K, _D), jnp.float32),
        pltpu.VMEM((_K, _D), jnp.float32),
        pltpu.VMEM_SHARED((_NP, _D), jnp.float32),
        [pltpu.SemaphoreType.DMA] * 2,
    ],
)
def _agg_kernel(h_hbm, src_hbm, dst_hbm, zeros_hbm, out_hbm, si_a, si_b,
                di_a, di_b, rows_a, rows_b, acc_sh, gsems):
    c = lax.axis_index("c")
    s = lax.axis_index("s")
    wid = c * _NS + s
    base = wid * _CPT
    r0 = s * _RPT
    pltpu.sync_copy(zeros_hbm.at[pl.ds(r0, _RPT)], acc_sh.at[pl.ds(r0, _RPT)])
    plsc.subcore_barrier()

    def e0(i):
        return pl.multiple_of((base + i) * _K, 8)

    def body(i, carry):
        e = e0(i)
        pltpu.sync_copy(src_hbm.at[pl.ds(e, _K)], si_a)
        pltpu.async_copy(h_hbm.at[si_a], rows_a, gsems[0]).wait()
        pltpu.sync_copy(dst_hbm.at[pl.ds(e, _K)], di_a.at[0])
        pltpu.sync_copy(rows_a, acc_sh.at[di_a.at[0]], add=True)
        return carry

    lax.fori_loop(0, _CPT, body, 0)

    plsc.subcore_barrier()
    pltpu.sync_copy(acc_sh.at[pl.ds(r0, _RPT)],
                    out_hbm.at[c].at[pl.ds(r0, _RPT)])


_NB = 2000
_GRID = _N // _NB


def _norm_body(deg_ref, x_ref, ns_ref, nd_ref, h0_ref):
    p = deg_ref[...]
    ds = p[0, :, 0] + p[1, :, 0]
    di = p[0, :, 64] + p[1, :, 64]
    ns = jnp.where(ds > 0, lax.rsqrt(ds), 0.0)[:, None]
    nd = jnp.where(di > 0, lax.rsqrt(di), 0.0)[:, None]
    ns_ref[...] = ns
    nd_ref[...] = nd
    h0_ref[...] = x_ref[...] * ns


def _norm_stage(deg_parts, x):
    return pl.pallas_call(
        _norm_body,
        grid=(_GRID,),
        in_specs=[
            pl.BlockSpec((_NC, _NB, _D), lambda i: (0, i, 0)),
            pl.BlockSpec((_NB, _D), lambda i: (i, 0)),
        ],
        out_specs=[
            pl.BlockSpec((_NB, 1), lambda i: (i, 0)),
            pl.BlockSpec((_NB, 1), lambda i: (i, 0)),
            pl.BlockSpec((_NB, _D), lambda i: (i, 0)),
        ],
        out_shape=[
            jax.ShapeDtypeStruct((_N, 1), jnp.float32),
            jax.ShapeDtypeStruct((_N, 1), jnp.float32),
            jax.ShapeDtypeStruct((_N, _D), jnp.float32),
        ],
    )(deg_parts, x)


def _layer_body(p_ref, nd_ref, w_ref, b_ref, ns_ref, o_ref):
    a = (p_ref[0] + p_ref[1]) * nd_ref[...]
    y = jnp.dot(a, w_ref[...], preferred_element_type=jnp.float32) + b_ref[...]
    y = jnp.where(y > 0, y, 0.01 * y)
    o_ref[...] = y * ns_ref[...]


def _layer_stage(parts, nd, w, b, ns):
    return pl.pallas_call(
        _layer_body,
        grid=(_GRID,),
        in_specs=[
            pl.BlockSpec((_NC, _NB, _D), lambda i: (0, i, 0)),
            pl.BlockSpec((_NB, 1), lambda i: (i, 0)),
            pl.BlockSpec((_D, _D), lambda i: (0, 0)),
            pl.BlockSpec((1, _D), lambda i: (0, 0)),
            pl.BlockSpec((_NB, 1), lambda i: (i, 0)),
        ],
        out_specs=pl.BlockSpec((_NB, _D), lambda i: (i, 0)),
        out_shape=jax.ShapeDtypeStruct((_N, _D), jnp.float32),
    )(parts, nd, w, b, ns)


def _final_body(p_ref, nd_ref, w_ref, b_ref, wl_ref, bl_ref, o_ref):
    a = (p_ref[0] + p_ref[1]) * nd_ref[...]
    y = jnp.dot(a, w_ref[...], preferred_element_type=jnp.float32) + b_ref[...]
    y = jnp.where(y > 0, y, 0.01 * y)
    o_ref[...] = (
        jnp.dot(y, wl_ref[...], preferred_element_type=jnp.float32) + bl_ref[...]
    )


def _final_stage(parts, nd, w, b, wl, bl):
    return pl.pallas_call(
        _final_body,
        grid=(_GRID,),
        in_specs=[
            pl.BlockSpec((_NC, _NB, _D), lambda i: (0, i, 0)),
            pl.BlockSpec((_NB, 1), lambda i: (i, 0)),
            pl.BlockSpec((_D, _D), lambda i: (0, 0)),
            pl.BlockSpec((1, _D), lambda i: (0, 0)),
            pl.BlockSpec((_D, _C), lambda i: (0, 0)),
            pl.BlockSpec((1, _C), lambda i: (0, 0)),
        ],
        out_specs=pl.BlockSpec((_NB, _C), lambda i: (i, 0)),
        out_shape=jax.ShapeDtypeStruct((_N, _C), jnp.float32),
    )(parts, nd, w, b, wl, bl)


def kernel(in_feat, edge_index, W1, b1, W2, b2, W3, b3, W4, b4, W5, b5, Wl, bl):
    src = edge_index[0]
    dst = edge_index[1]
    npad = _EPAD - _E
    pad0 = jnp.zeros((npad,), jnp.int32)
    padt = _N + (jnp.arange(npad, dtype=jnp.int32) % (_NP - _N))
    src_agg = jnp.concatenate([src, pad0])
    src_deg = jnp.concatenate([src, padt]).reshape(_NW, _CPT, _K)
    dst_pad = jnp.concatenate([dst, padt])
    dst_deg = dst_pad.reshape(_NW, _CPT, _K)
    zeros_big = jnp.zeros((_NP, _D), jnp.float32)
    lanes = jnp.arange(_D) < 64
    ones_s = jnp.broadcast_to(lanes.astype(jnp.float32), (_K, _D))
    ones_d = jnp.broadcast_to((~lanes).astype(jnp.float32), (_K, _D))

    deg_parts = _degree_kernel(src_deg, dst_deg, zeros_big, ones_s, ones_d)
    ns, nd, h = _norm_stage(deg_parts, in_feat)
    for w, b in ((W1, b1), (W2, b2), (W3, b3), (W4, b4)):
        parts = _agg_kernel(h, src_agg, dst_pad, zeros_big)
        h = _layer_stage(parts, nd, w, b.reshape(1, _D), ns)
    parts = _agg_kernel(h, src_agg, dst_pad, zeros_big)
    return _final_stage(parts, nd, W5, b5.reshape(1, _D), Wl, bl.reshape(1, _C))


# speedup vs baseline: 2.1889x; 2.1889x over previous
import functools

import jax
import jax.numpy as jnp
from jax import lax
from jax.experimental import pallas as pl
from jax.experimental.pallas import tpu as pltpu
from jax.experimental.pallas import tpu_sc as plsc

_N = 10000
_E = 320000
_D = 128
_C = 16

_NC = 2
_NS = 16
_NW = _NC * _NS
_K = 128
_CPT = 80
_EPAD = _NW * _CPT * _K
_NP = 10240
_TRASH = 10200
_RPT = _NP // _NS
_NBUF = 2
_J = 8
_NJ = _CPT // _J

_mesh = plsc.VectorSubcoreMesh(
    core_axis_name="c", subcore_axis_name="s", num_cores=_NC, num_subcores=_NS
)


def _load_block(src_hbm, dst_hbm, si2, di2, bsems, wid, j, sl):
    j0 = pl.multiple_of(j * _J, 8)
    pltpu.async_copy(src_hbm.at[wid].at[pl.ds(j0, _J)], si2.at[sl], bsems[sl])
    pltpu.async_copy(dst_hbm.at[wid].at[pl.ds(j0, _J)], di2.at[sl], bsems[sl])


def _wait_block(src_hbm, dst_hbm, si2, di2, bsems, wid, j, sl):
    j0 = pl.multiple_of(j * _J, 8)
    pltpu.make_async_copy(
        src_hbm.at[wid].at[pl.ds(j0, _J)], si2.at[sl], bsems[sl]).wait()
    pltpu.make_async_copy(
        dst_hbm.at[wid].at[pl.ds(j0, _J)], di2.at[sl], bsems[sl]).wait()


@functools.partial(
    pl.kernel,
    out_type=jax.ShapeDtypeStruct((_NC, _NP, _D), jnp.float32),
    mesh=_mesh,
    scratch_types=[
        pltpu.VMEM((2, _J, _K), jnp.int32),
        pltpu.VMEM((2, _J, _K), jnp.int32),
        pltpu.VMEM((_K, _D), jnp.float32),
        pltpu.VMEM((_K, _D), jnp.float32),
        pltpu.VMEM_SHARED((_NP, _D), jnp.float32),
        [pltpu.SemaphoreType.DMA] * 2,
        pltpu.SemaphoreType.DMA,
    ],
)
def _degree_kernel(src_hbm, dst_hbm, zeros_hbm, ones_s_hbm, ones_d_hbm,
                   out_hbm, si2, di2, ones_s, ones_d, acc, bsems, ssem):
    c = lax.axis_index("c")
    s = lax.axis_index("s")
    wid = c * _NS + s
    r0 = s * _RPT
    pltpu.sync_copy(ones_s_hbm, ones_s)
    pltpu.sync_copy(ones_d_hbm, ones_d)
    pltpu.sync_copy(zeros_hbm.at[pl.ds(r0, _RPT)], acc.at[pl.ds(r0, _RPT)])
    plsc.subcore_barrier()

    def emit_block(j, p, wait_next, load_next):
        for q in range(_J):
            pltpu.async_copy(ones_s, acc.at[si2.at[p].at[q]], ssem, add=True)
            pltpu.async_copy(ones_d, acc.at[di2.at[p].at[q]], ssem, add=True)
        for q in range(2 * _J):
            pltpu.make_async_copy(ones_s, acc.at[si2.at[p].at[0]],
                                  ssem).wait()
        if wait_next:
            _wait_block(src_hbm, dst_hbm, si2, di2, bsems, wid, j + 1, 1 - p)
        if load_next:
            _load_block(src_hbm, dst_hbm, si2, di2, bsems, wid, j + 2, p)

    _load_block(src_hbm, dst_hbm, si2, di2, bsems, wid, 0, 0)
    _load_block(src_hbm, dst_hbm, si2, di2, bsems, wid, 1, 1)
    _wait_block(src_hbm, dst_hbm, si2, di2, bsems, wid, 0, 0)

    def body(j0, carry):
        emit_block(j0 * 2, 0, True, True)
        emit_block(j0 * 2 + 1, 1, True, True)
        return carry

    lax.fori_loop(0, (_NJ - 2) // 2, body, 0)
    emit_block(_NJ - 2, 0, True, False)
    emit_block(_NJ - 1, 1, False, False)

    plsc.subcore_barrier()
    pltpu.sync_copy(acc.at[pl.ds(r0, _RPT)],
                    out_hbm.at[c].at[pl.ds(r0, _RPT)])


@functools.partial(
    pl.kernel,
    out_type=jax.ShapeDtypeStruct((_NC, _NP, _D), jnp.float32),
    mesh=_mesh,
    scratch_types=[
        pltpu.VMEM((_K,), jnp.int32),
        pltpu.VMEM((_K,), jnp.int32),
        pltpu.VMEM((1, _K), jnp.int32),
        pltpu.VMEM((1, _K), jnp.int32),
        pltpu.VMEM((_K, _D), jnp.float32),
        pltpu.VMEM((_K, _D), jnp.float32),
        pltpu.VMEM_SHARED((_NP, _D), jnp.float32),
        [pltpu.SemaphoreType.DMA] * 2,
    ],
)
def _agg_kernel(h_hbm, src_hbm, dst_hbm, zeros_hbm, out_hbm, si_a, si_b,
                di_a, di_b, rows_a, rows_b, acc_sh, gsems):
    c = lax.axis_index("c")
    s = lax.axis_index("s")
    wid = c * _NS + s
    r0 = s * _RPT
    pltpu.sync_copy(zeros_hbm.at[pl.ds(r0, _RPT)], acc_sh.at[pl.ds(r0, _RPT)])
    plsc.subcore_barrier()

    def body(i, carry):
        j = wid + i * _NW
        e = pl.multiple_of(j * _K, 8)
        pltpu.sync_copy(src_hbm.at[pl.ds(e, _K)], si_a)
        pltpu.async_copy(h_hbm.at[si_a], rows_a, gsems[0]).wait()
        pltpu.sync_copy(dst_hbm.at[pl.ds(e, _K)], di_a.at[0])
        pltpu.sync_copy(rows_a, acc_sh.at[di_a.at[0]], add=True)
        return carry

    nchunks = (_E // _K) // _NW + jnp.where(wid < (_E // _K) % _NW, 1, 0)
    lax.fori_loop(0, nchunks, body, 0)

    plsc.subcore_barrier()
    pltpu.sync_copy(acc_sh.at[pl.ds(r0, _RPT)],
                    out_hbm.at[c].at[pl.ds(r0, _RPT)])


_NB = 2000
_GRID = _N // _NB


def _norm_body(deg_ref, x_ref, ns_ref, nd_ref, h0_ref):
    p = deg_ref[...]
    ds = p[0, :, 0] + p[1, :, 0]
    di = p[0, :, 64] + p[1, :, 64]
    ns = jnp.where(ds > 0, lax.rsqrt(ds), 0.0)[:, None]
    nd = jnp.where(di > 0, lax.rsqrt(di), 0.0)[:, None]
    ns_ref[...] = ns
    nd_ref[...] = nd
    h0_ref[...] = x_ref[...] * ns


def _norm_stage(deg_parts, x):
    return pl.pallas_call(
        _norm_body,
        grid=(_GRID,),
        in_specs=[
            pl.BlockSpec((_NC, _NB, _D), lambda i: (0, i, 0)),
            pl.BlockSpec((_NB, _D), lambda i: (i, 0)),
        ],
        out_specs=[
            pl.BlockSpec((_NB, 1), lambda i: (i, 0)),
            pl.BlockSpec((_NB, 1), lambda i: (i, 0)),
            pl.BlockSpec((_NB, _D), lambda i: (i, 0)),
        ],
        out_shape=[
            jax.ShapeDtypeStruct((_N, 1), jnp.float32),
            jax.ShapeDtypeStruct((_N, 1), jnp.float32),
            jax.ShapeDtypeStruct((_N, _D), jnp.float32),
        ],
    )(deg_parts, x)


def _layer_body(p_ref, nd_ref, w_ref, b_ref, ns_ref, o_ref):
    a = (p_ref[0] + p_ref[1]) * nd_ref[...]
    y = jnp.dot(a, w_ref[...], preferred_element_type=jnp.float32) + b_ref[...]
    y = jnp.where(y > 0, y, 0.01 * y)
    o_ref[...] = y * ns_ref[...]


def _layer_stage(parts, nd, w, b, ns):
    return pl.pallas_call(
        _layer_body,
        grid=(_GRID,),
        in_specs=[
            pl.BlockSpec((_NC, _NB, _D), lambda i: (0, i, 0)),
            pl.BlockSpec((_NB, 1), lambda i: (i, 0)),
            pl.BlockSpec((_D, _D), lambda i: (0, 0)),
            pl.BlockSpec((1, _D), lambda i: (0, 0)),
            pl.BlockSpec((_NB, 1), lambda i: (i, 0)),
        ],
        out_specs=pl.BlockSpec((_NB, _D), lambda i: (i, 0)),
        out_shape=jax.ShapeDtypeStruct((_N, _D), jnp.float32),
    )(parts, nd, w, b, ns)


def _final_body(p_ref, nd_ref, w_ref, b_ref, wl_ref, bl_ref, o_ref):
    a = (p_ref[0] + p_ref[1]) * nd_ref[...]
    y = jnp.dot(a, w_ref[...], preferred_element_type=jnp.float32) + b_ref[...]
    y = jnp.where(y > 0, y, 0.01 * y)
    o_ref[...] = (
        jnp.dot(y, wl_ref[...], preferred_element_type=jnp.float32) + bl_ref[...]
    )


def _final_stage(parts, nd, w, b, wl, bl):
    return pl.pallas_call(
        _final_body,
        grid=(_GRID,),
        in_specs=[
            pl.BlockSpec((_NC, _NB, _D), lambda i: (0, i, 0)),
            pl.BlockSpec((_NB, 1), lambda i: (i, 0)),
            pl.BlockSpec((_D, _D), lambda i: (0, 0)),
            pl.BlockSpec((1, _D), lambda i: (0, 0)),
            pl.BlockSpec((_D, _C), lambda i: (0, 0)),
            pl.BlockSpec((1, _C), lambda i: (0, 0)),
        ],
        out_specs=pl.BlockSpec((_NB, _C), lambda i: (i, 0)),
        out_shape=jax.ShapeDtypeStruct((_N, _C), jnp.float32),
    )(parts, nd, w, b, wl, bl)


def kernel(in_feat, edge_index, W1, b1, W2, b2, W3, b3, W4, b4, W5, b5, Wl, bl):
    src = edge_index[0]
    dst = edge_index[1]
    npad = _EPAD - _E
    pad0 = jnp.zeros((npad,), jnp.int32)
    padt = _N + (jnp.arange(npad, dtype=jnp.int32) % (_NP - _N))
    src_agg = jnp.concatenate([src, pad0])
    src_deg = jnp.concatenate([src, padt]).reshape(_NW, _CPT, _K)
    dst_pad = jnp.concatenate([dst, padt])
    dst_deg = dst_pad.reshape(_NW, _CPT, _K)
    zeros_big = jnp.zeros((_NP, _D), jnp.float32)
    lanes = jnp.arange(_D) < 64
    ones_s = jnp.broadcast_to(lanes.astype(jnp.float32), (_K, _D))
    ones_d = jnp.broadcast_to((~lanes).astype(jnp.float32), (_K, _D))

    deg_parts = _degree_kernel(src_deg, dst_deg, zeros_big, ones_s, ones_d)
    ns, nd, h = _norm_stage(deg_parts, in_feat)
    for w, b in ((W1, b1), (W2, b2), (W3, b3), (W4, b4)):
        parts = _agg_kernel(h, src_agg, dst_pad, zeros_big)
        h = _layer_stage(parts, nd, w, b.reshape(1, _D), ns)
    parts = _agg_kernel(h, src_agg, dst_pad, zeros_big)
    return _final_stage(parts, nd, W5, b5.reshape(1, _D), Wl, bl.reshape(1, _C))
